# baseline (speedup 1.0000x reference)
import functools

import jax
import jax.numpy as jnp
from jax import lax
from jax.experimental import pallas as pl
from jax.experimental.pallas import tpu as pltpu
from jax.experimental.pallas import tpu_sc as plsc

_N = 50000
_E = 1600000
_NSUB = 16
_NCORE = 2
_NW = _NCORE * _NSUB
_EPW = 51200
_EPAD = _NW * _EPW
_NPAD = 50176
_RPS = _NPAD // _NSUB
_NBLK = 8
_P32 = _NPAD * 32 // 128
_P16 = _NPAD * 16 // 128
_B32 = _P32 // _NBLK
_B16 = _P16 // _NBLK

_C32, _NB32 = 400, 1
_C16, _NB16 = 800, 2
_CDEG, _NBDEG = 1600, 2


def _mesh():
    return plsc.VectorSubcoreMesh(core_axis_name="c", subcore_axis_name="s")


_SC_PARAMS = pltpu.CompilerParams(use_tc_tiling_on_sc=False)


@functools.partial(
    pl.kernel,
    out_type=jax.ShapeDtypeStruct((2, _NPAD, 16), jnp.float32),
    mesh=_mesh(),
    scratch_types=[
        pltpu.VMEM((2, _NBDEG, _CDEG), jnp.int32),
        pltpu.VMEM((_CDEG, 16), jnp.float32),
        pltpu.VMEM_SHARED((_NPAD, 16), jnp.float32),
        pltpu.SemaphoreType.DMA,
    ],
    compiler_params=_SC_PARAMS,
)
def _sc_degree(dst_hbm, ones_hbm, zeros_hbm, out_hbm, dst_v, ones_v, acc, ssem):
    cid = lax.axis_index("c")
    sid = lax.axis_index("s")
    wid = cid * _NSUB + sid
    row0 = sid * _RPS
    ch = _EPW // _CDEG
    groups = ch // _NBDEG
    pltpu.sync_copy(ones_hbm, ones_v)
    pltpu.sync_copy(zeros_hbm.at[pl.ds(row0, _RPS)], acc.at[pl.ds(row0, _RPS)])
    plsc.subcore_barrier()
    chunk0 = wid * ch
    pltpu.sync_copy(dst_hbm.at[pl.ds(chunk0, _NBDEG)], dst_v.at[0])

    @pl.loop(0, groups, step=2)
    def _(g0):
        for off in range(2):
            g = g0 + off
            slot, other = off, 1 - off

            @pl.when(g > 0)
            def _():
                for b in range(_NBDEG):
                    pltpu.make_async_copy(
                        ones_v, acc.at[dst_v.at[other, b]], ssem).wait()

            @pl.when(g + 1 < groups)
            def _():
                pltpu.sync_copy(
                    dst_hbm.at[pl.ds(chunk0 + (g + 1) * _NBDEG, _NBDEG)],
                    dst_v.at[other])

            for b in range(_NBDEG):
                pltpu.async_copy(ones_v, acc.at[dst_v.at[slot, b]], ssem, add=True)

    for b in range(_NBDEG):
        pltpu.make_async_copy(ones_v, acc.at[dst_v.at[1, b]], ssem).wait()

    plsc.subcore_barrier()

    @pl.when(cid == 0)
    def _():
        pltpu.sync_copy(acc.at[pl.ds(row0, _RPS)], out_hbm.at[0, pl.ds(row0, _RPS)])

    @pl.when(cid == 1)
    def _():
        pltpu.sync_copy(acc.at[pl.ds(row0, _RPS)], out_hbm.at[1, pl.ds(row0, _RPS)])


def _make_sc_scatter(dp, c, nb):
    ch = _EPW // c
    groups = ch // nb

    @functools.partial(
        pl.kernel,
        out_type=jax.ShapeDtypeStruct((2, _NPAD, dp), jnp.float32),
        mesh=_mesh(),
        scratch_types=[
            pltpu.VMEM((2, nb, c), jnp.int32),
            pltpu.VMEM((2, nb, c), jnp.int32),
            [[pltpu.VMEM((c, dp), jnp.float32) for _ in range(nb)]
             for _ in range(2)],
            pltpu.VMEM_SHARED((_NPAD, dp), jnp.float32),
            pltpu.SemaphoreType.DMA,
            pltpu.SemaphoreType.DMA,
        ],
        compiler_params=_SC_PARAMS,
    )
    def k(g_hbm, src_hbm, dst_hbm, zeros_hbm, out_hbm,
          src_v, dst_v, rows, acc, gsem, ssem):
        cid = lax.axis_index("c")
        sid = lax.axis_index("s")
        wid = cid * _NSUB + sid
        row0 = sid * _RPS
        pltpu.sync_copy(zeros_hbm.at[pl.ds(row0, _RPS)], acc.at[pl.ds(row0, _RPS)])
        plsc.subcore_barrier()
        chunk0 = wid * ch
        pltpu.sync_copy(src_hbm.at[pl.ds(chunk0, nb)], src_v.at[0])
        pltpu.sync_copy(dst_hbm.at[pl.ds(chunk0, nb)], dst_v.at[0])
        for b in range(nb):
            pltpu.async_copy(g_hbm.at[src_v.at[0, b]], rows[0][b], gsem)

        @pl.loop(0, groups, step=2)
        def _(g0):
            for off in range(2):
                g = g0 + off
                slot, other = off, 1 - off

                @pl.when(g > 0)
                def _():
                    for b in range(nb):
                        pltpu.make_async_copy(
                            rows[other][b], acc.at[dst_v.at[other, b]], ssem).wait()

                @pl.when(g + 1 < groups)
                def _():
                    base = chunk0 + (g + 1) * nb
                    pltpu.sync_copy(src_hbm.at[pl.ds(base, nb)], src_v.at[other])
                    pltpu.sync_copy(dst_hbm.at[pl.ds(base, nb)], dst_v.at[other])
                    for b in range(nb):
                        pltpu.async_copy(
                            g_hbm.at[src_v.at[other, b]], rows[other][b], gsem)

                for b in range(nb):
                    pltpu.make_async_copy(
                        g_hbm.at[src_v.at[slot, b]], rows[slot][b], gsem).wait()
                    pltpu.async_copy(
                        rows[slot][b], acc.at[dst_v.at[slot, b]], ssem, add=True)

        for b in range(nb):
            pltpu.make_async_copy(rows[1][b], acc.at[dst_v.at[1, b]], ssem).wait()

        plsc.subcore_barrier()

        @pl.when(cid == 0)
        def _():
            pltpu.sync_copy(acc.at[pl.ds(row0, _RPS)], out_hbm.at[0, pl.ds(row0, _RPS)])

        @pl.when(cid == 1)
        def _():
            pltpu.sync_copy(acc.at[pl.ds(row0, _RPS)], out_hbm.at[1, pl.ds(row0, _RPS)])

    return k


_sc_scatter32 = _make_sc_scatter(32, _C32, _NB32)
_sc_scatter16 = _make_sc_scatter(16, _C16, _NB16)


def _t1_body(da_ref, db_ref, xp_ref, w_ref, me_ref, mo_ref,
             g_ref, d32_ref, d16_ref):
    dinv16 = lax.rsqrt(da_ref[...] + db_ref[...] + 1.0)
    d16_ref[...] = dinv16
    de = jnp.dot(dinv16, me_ref[...], preferred_element_type=jnp.float32)
    do = jnp.dot(dinv16, mo_ref[...], preferred_element_type=jnp.float32)
    dinv32 = jnp.stack([de, do], axis=1).reshape(_B32, 128)
    d32_ref[...] = dinv32
    h = jnp.dot(xp_ref[...], w_ref[...], preferred_element_type=jnp.float32)
    g_ref[...] = dinv32 * h


def _t1(deg2p, xpp, w_bd, me, mo):
    return pl.pallas_call(
        _t1_body,
        grid=(_NBLK,),
        in_specs=[
            pl.BlockSpec((_B16, 128), lambda i: (i, 0)),
            pl.BlockSpec((_B16, 128), lambda i: (i + _NBLK, 0)),
            pl.BlockSpec((_B32, 128), lambda i: (i, 0)),
            pl.BlockSpec((128, 128), lambda i: (0, 0)),
            pl.BlockSpec((128, 128), lambda i: (0, 0)),
            pl.BlockSpec((128, 128), lambda i: (0, 0)),
        ],
        out_specs=[
            pl.BlockSpec((_B32, 128), lambda i: (i, 0)),
            pl.BlockSpec((_B32, 128), lambda i: (i, 0)),
            pl.BlockSpec((_B16, 128), lambda i: (i, 0)),
        ],
        out_shape=[
            jax.ShapeDtypeStruct((_P32, 128), jnp.float32),
            jax.ShapeDtypeStruct((_P32, 128), jnp.float32),
            jax.ShapeDtypeStruct((_P16, 128), jnp.float32),
        ],
    )(deg2p, deg2p, xpp, w_bd, me, mo)


def _tmid_same_body(sa_ref, sb_ref, g_ref, dz_ref, b_ref, w_ref, go_ref):
    act = jnp.maximum(
        dz_ref[...] * (sa_ref[0] + sb_ref[0] + g_ref[...]) + b_ref[...], 0.0)
    go_ref[...] = jnp.dot(dz_ref[...] * act, w_ref[...],
                          preferred_element_type=jnp.float32)


def _tmid_same(sp, g, dz, b, w_bd, pin):
    bi = pin // _NBLK
    sp2 = sp.reshape(2, pin, 128)
    return pl.pallas_call(
        _tmid_same_body,
        grid=(_NBLK,),
        in_specs=[
            pl.BlockSpec((1, bi, 128), lambda i: (0, i, 0)),
            pl.BlockSpec((1, bi, 128), lambda i: (1, i, 0)),
            pl.BlockSpec((bi, 128), lambda i: (i, 0)),
            pl.BlockSpec((bi, 128), lambda i: (i, 0)),
            pl.BlockSpec((1, 128), lambda i: (0, 0)),
            pl.BlockSpec((128, 128), lambda i: (0, 0)),
        ],
        out_specs=pl.BlockSpec((bi, 128), lambda i: (i, 0)),
        out_shape=jax.ShapeDtypeStruct((pin, 128), jnp.float32),
    )(sp2, sp2, g, dz, b, w_bd)


def _t3_body(sa_ref, sb_ref, g_ref, dz_ref, b_ref, w_ref, go_ref):
    act = jnp.maximum(
        dz_ref[...] * (sa_ref[0] + sb_ref[0] + g_ref[...]) + b_ref[...], 0.0)
    scaled = (dz_ref[...] * act).reshape(_B16, 2, 128)
    he = jnp.dot(scaled[:, 0, :], w_ref[...],
                 preferred_element_type=jnp.float32)
    ho = jnp.dot(scaled[:, 1, :], w_ref[...],
                 preferred_element_type=jnp.float32)
    go_ref[...] = jnp.concatenate([he, ho], axis=1)


def _t3(sp, g, dz, b, w_bd):
    sp2 = sp.reshape(2, _P32, 128)
    return pl.pallas_call(
        _t3_body,
        grid=(_NBLK,),
        in_specs=[
            pl.BlockSpec((1, _B32, 128), lambda i: (0, i, 0)),
            pl.BlockSpec((1, _B32, 128), lambda i: (1, i, 0)),
            pl.BlockSpec((_B32, 128), lambda i: (i, 0)),
            pl.BlockSpec((_B32, 128), lambda i: (i, 0)),
            pl.BlockSpec((1, 128), lambda i: (0, 0)),
            pl.BlockSpec((128, 64), lambda i: (0, 0)),
        ],
        out_specs=pl.BlockSpec((_B16, 128), lambda i: (i, 0)),
        out_shape=jax.ShapeDtypeStruct((_P16, 128), jnp.float32),
    )(sp2, sp2, g, dz, b, w_bd)


def _tfin_body(sa_ref, sb_ref, g_ref, d16_ref, b_ref, gsum_ref, out_ref):
    z = jnp.maximum(
        d16_ref[...] * (sa_ref[0] + sb_ref[0] + g_ref[...]) + b_ref[...], 0.0)
    col = lax.broadcasted_iota(jnp.int32, z.shape, 1) % 16
    e = jnp.exp(z) * jnp.where(col < 5, 1.0, 0.0)
    denom = jnp.dot(e, gsum_ref[...], preferred_element_type=jnp.float32)
    out_ref[...] = e / denom


def _tfin(sp, g, d16, b, gsum):
    sp2 = sp.reshape(2, _P16, 128)
    return pl.pallas_call(
        _tfin_body,
        grid=(_NBLK,),
        in_specs=[
            pl.BlockSpec((1, _B16, 128), lambda i: (0, i, 0)),
            pl.BlockSpec((1, _B16, 128), lambda i: (1, i, 0)),
            pl.BlockSpec((_B16, 128), lambda i: (i, 0)),
            pl.BlockSpec((_B16, 128), lambda i: (i, 0)),
            pl.BlockSpec((1, 128), lambda i: (0, 0)),
            pl.BlockSpec((128, 128), lambda i: (0, 0)),
        ],
        out_specs=pl.BlockSpec((_B16, 128), lambda i: (i, 0)),
        out_shape=jax.ShapeDtypeStruct((_P16, 128), jnp.float32),
    )(sp2, sp2, g, d16, b, gsum)


def _pad2(a, r, c):
    out = jnp.zeros((r, c), dtype=jnp.float32)
    return out.at[: a.shape[0], : a.shape[1]].set(a)


def kernel(x, edge_index, W1, b1, W3, b3, W4, b4, W5, b5, W2, b2):
    ei = edge_index.astype(jnp.int32)
    pad_ids = _N + (jnp.arange(_EPAD - _E, dtype=jnp.int32) % 128)
    src_p = jnp.concatenate([ei[0], pad_ids])
    dst_p = jnp.concatenate([ei[1], pad_ids])
    src32 = src_p.reshape(-1, _C32)
    dst32 = dst_p.reshape(-1, _C32)
    src16 = src_p.reshape(-1, _C16)
    dst16 = dst_p.reshape(-1, _C16)
    dstdeg = dst_p.reshape(-1, _CDEG)

    xpp = _pad2(x, _NPAD, 32).reshape(_P32, 128)
    w1_bd = jnp.kron(jnp.eye(4, dtype=jnp.float32), _pad2(W1, 32, 32))
    w3_bd = jnp.kron(jnp.eye(4, dtype=jnp.float32), _pad2(W3, 32, 32))
    w4_bd = jnp.kron(jnp.eye(4, dtype=jnp.float32), _pad2(W4, 32, 16))
    w5_bd = jnp.kron(jnp.eye(8, dtype=jnp.float32), _pad2(W5, 16, 16))
    w2_bd = jnp.kron(jnp.eye(8, dtype=jnp.float32), _pad2(W2, 16, 16))
    b1t = jnp.tile(_pad2(b1[None, :], 1, 32), (1, 4))
    b3t = jnp.tile(_pad2(b3[None, :], 1, 32), (1, 4))
    b4t = jnp.tile(_pad2(b4[None, :], 1, 16), (1, 8))
    b5t = jnp.tile(_pad2(b5[None, :], 1, 16), (1, 8))
    b2t = jnp.tile(_pad2(b2[None, :], 1, 16), (1, 8))

    lane = jnp.arange(128)
    m_even = (lane[:, None] == 16 * (lane[None, :] // 32)).astype(jnp.float32)
    m_odd = (lane[:, None] == 64 + 16 * (lane[None, :] // 32)).astype(jnp.float32)
    gsum = (lane[:, None] // 16 == lane[None, :] // 16).astype(jnp.float32)

    zeros32 = jnp.zeros((_NPAD, 32), jnp.float32)
    zeros16 = jnp.zeros((_NPAD, 16), jnp.float32)
    ones16 = jnp.ones((_CDEG, 16), jnp.float32)

    deg2 = _sc_degree(dstdeg, ones16, zeros16)
    g1, d32, d16 = _t1(deg2.reshape(2 * _P16, 128), xpp, w1_bd, m_even, m_odd)
    s1 = _sc_scatter32(g1.reshape(_NPAD, 32), src32, dst32, zeros32)
    g2 = _tmid_same(s1, g1, d32, b1t, w3_bd, _P32)
    s2 = _sc_scatter32(g2.reshape(_NPAD, 32), src32, dst32, zeros32)
    g3 = _t3(s2, g2, d32, b3t, w4_bd)
    s3 = _sc_scatter16(g3.reshape(_NPAD, 16), src16, dst16, zeros16)
    g4 = _tmid_same(s3, g3, d16, b4t, w5_bd, _P16)
    s4 = _sc_scatter16(g4.reshape(_NPAD, 16), src16, dst16, zeros16)
    g5 = _tmid_same(s4, g4, d16, b5t, w2_bd, _P16)
    s5 = _sc_scatter16(g5.reshape(_NPAD, 16), src16, dst16, zeros16)
    p = _tfin(s5, g5, d16, b2t, gsum)
    return p.reshape(_NPAD, 16)[:_N, :5]


# speedup vs baseline: 1.0102x; 1.0102x over previous
import functools

import jax
import jax.numpy as jnp
from jax import lax
from jax.experimental import pallas as pl
from jax.experimental.pallas import tpu as pltpu
from jax.experimental.pallas import tpu_sc as plsc

_N = 50000
_E = 1600000
_NSUB = 16
_NCORE = 2
_NW = _NCORE * _NSUB
_EPW = 51200
_EPAD = _NW * _EPW
_NPAD = 50176
_RPS = _NPAD // _NSUB
_NBLK = 8
_P32 = _NPAD * 32 // 128
_P16 = _NPAD * 16 // 128
_B32 = _P32 // _NBLK
_B16 = _P16 // _NBLK

_C32, _NB32 = 400, 1
_C16, _NB16 = 400, 4
_CDEG, _NBDEG = 800, 4


def _mesh():
    return plsc.VectorSubcoreMesh(core_axis_name="c", subcore_axis_name="s")


_SC_PARAMS = pltpu.CompilerParams(use_tc_tiling_on_sc=False)


@functools.partial(
    pl.kernel,
    out_type=jax.ShapeDtypeStruct((2, _NPAD, 16), jnp.float32),
    mesh=_mesh(),
    scratch_types=[
        pltpu.VMEM((2, _NBDEG, _CDEG), jnp.int32),
        pltpu.VMEM((_CDEG, 16), jnp.float32),
        pltpu.VMEM_SHARED((_NPAD, 16), jnp.float32),
        pltpu.SemaphoreType.DMA,
    ],
    compiler_params=_SC_PARAMS,
)
def _sc_degree(dst_hbm, ones_hbm, zeros_hbm, out_hbm, dst_v, ones_v, acc, ssem):
    cid = lax.axis_index("c")
    sid = lax.axis_index("s")
    wid = cid * _NSUB + sid
    row0 = sid * _RPS
    ch = _EPW // _CDEG
    groups = ch // _NBDEG
    pltpu.sync_copy(ones_hbm, ones_v)
    pltpu.sync_copy(zeros_hbm.at[pl.ds(row0, _RPS)], acc.at[pl.ds(row0, _RPS)])
    plsc.subcore_barrier()
    chunk0 = wid * ch
    pltpu.sync_copy(dst_hbm.at[pl.ds(chunk0, _NBDEG)], dst_v.at[0])

    @pl.loop(0, groups, step=2)
    def _(g0):
        for off in range(2):
            g = g0 + off
            slot, other = off, 1 - off

            @pl.when(g > 0)
            def _():
                for b in range(_NBDEG):
                    pltpu.make_async_copy(
                        ones_v, acc.at[dst_v.at[other, b]], ssem).wait()

            @pl.when(g + 1 < groups)
            def _():
                pltpu.sync_copy(
                    dst_hbm.at[pl.ds(chunk0 + (g + 1) * _NBDEG, _NBDEG)],
                    dst_v.at[other])

            for b in range(_NBDEG):
                pltpu.async_copy(ones_v, acc.at[dst_v.at[slot, b]], ssem, add=True)

    for b in range(_NBDEG):
        pltpu.make_async_copy(ones_v, acc.at[dst_v.at[1, b]], ssem).wait()

    plsc.subcore_barrier()

    @pl.when(cid == 0)
    def _():
        pltpu.sync_copy(acc.at[pl.ds(row0, _RPS)], out_hbm.at[0, pl.ds(row0, _RPS)])

    @pl.when(cid == 1)
    def _():
        pltpu.sync_copy(acc.at[pl.ds(row0, _RPS)], out_hbm.at[1, pl.ds(row0, _RPS)])


def _make_sc_scatter(dp, c, nb):
    ch = _EPW // c
    groups = ch // nb

    @functools.partial(
        pl.kernel,
        out_type=jax.ShapeDtypeStruct((2, _NPAD, dp), jnp.float32),
        mesh=_mesh(),
        scratch_types=[
            pltpu.VMEM((2, nb, c), jnp.int32),
            pltpu.VMEM((2, nb, c), jnp.int32),
            [[pltpu.VMEM((c, dp), jnp.float32) for _ in range(nb)]
             for _ in range(2)],
            pltpu.VMEM_SHARED((_NPAD, dp), jnp.float32),
            pltpu.SemaphoreType.DMA,
            pltpu.SemaphoreType.DMA,
        ],
        compiler_params=_SC_PARAMS,
    )
    def k(g_hbm, src_hbm, dst_hbm, zeros_hbm, out_hbm,
          src_v, dst_v, rows, acc, gsem, ssem):
        cid = lax.axis_index("c")
        sid = lax.axis_index("s")
        wid = cid * _NSUB + sid
        row0 = sid * _RPS
        pltpu.sync_copy(zeros_hbm.at[pl.ds(row0, _RPS)], acc.at[pl.ds(row0, _RPS)])
        plsc.subcore_barrier()
        chunk0 = wid * ch
        pltpu.sync_copy(src_hbm.at[pl.ds(chunk0, nb)], src_v.at[0])
        pltpu.sync_copy(dst_hbm.at[pl.ds(chunk0, nb)], dst_v.at[0])
        for b in range(nb):
            pltpu.async_copy(g_hbm.at[src_v.at[0, b]], rows[0][b], gsem)

        @pl.loop(0, groups, step=2)
        def _(g0):
            for off in range(2):
                g = g0 + off
                slot, other = off, 1 - off

                @pl.when(g > 0)
                def _():
                    for b in range(nb):
                        pltpu.make_async_copy(
                            rows[other][b], acc.at[dst_v.at[other, b]], ssem).wait()

                @pl.when(g + 1 < groups)
                def _():
                    base = chunk0 + (g + 1) * nb
                    pltpu.sync_copy(src_hbm.at[pl.ds(base, nb)], src_v.at[other])
                    pltpu.sync_copy(dst_hbm.at[pl.ds(base, nb)], dst_v.at[other])
                    for b in range(nb):
                        pltpu.async_copy(
                            g_hbm.at[src_v.at[other, b]], rows[other][b], gsem)

                for b in range(nb):
                    pltpu.make_async_copy(
                        g_hbm.at[src_v.at[slot, b]], rows[slot][b], gsem).wait()
                    pltpu.async_copy(
                        rows[slot][b], acc.at[dst_v.at[slot, b]], ssem, add=True)

        for b in range(nb):
            pltpu.make_async_copy(rows[1][b], acc.at[dst_v.at[1, b]], ssem).wait()

        plsc.subcore_barrier()

        @pl.when(cid == 0)
        def _():
            pltpu.sync_copy(acc.at[pl.ds(row0, _RPS)], out_hbm.at[0, pl.ds(row0, _RPS)])

        @pl.when(cid == 1)
        def _():
            pltpu.sync_copy(acc.at[pl.ds(row0, _RPS)], out_hbm.at[1, pl.ds(row0, _RPS)])

    return k


_sc_scatter32 = _make_sc_scatter(32, _C32, _NB32)
_sc_scatter16 = _make_sc_scatter(16, _C16, _NB16)


def _t1_body(da_ref, db_ref, xp_ref, w_ref, me_ref, mo_ref,
             g_ref, d32_ref, d16_ref):
    dinv16 = lax.rsqrt(da_ref[...] + db_ref[...] + 1.0)
    d16_ref[...] = dinv16
    de = jnp.dot(dinv16, me_ref[...], preferred_element_type=jnp.float32)
    do = jnp.dot(dinv16, mo_ref[...], preferred_element_type=jnp.float32)
    dinv32 = jnp.stack([de, do], axis=1).reshape(_B32, 128)
    d32_ref[...] = dinv32
    h = jnp.dot(xp_ref[...], w_ref[...], preferred_element_type=jnp.float32)
    g_ref[...] = dinv32 * h


def _t1(deg2p, xpp, w_bd, me, mo):
    return pl.pallas_call(
        _t1_body,
        grid=(_NBLK,),
        in_specs=[
            pl.BlockSpec((_B16, 128), lambda i: (i, 0)),
            pl.BlockSpec((_B16, 128), lambda i: (i + _NBLK, 0)),
            pl.BlockSpec((_B32, 128), lambda i: (i, 0)),
            pl.BlockSpec((128, 128), lambda i: (0, 0)),
            pl.BlockSpec((128, 128), lambda i: (0, 0)),
            pl.BlockSpec((128, 128), lambda i: (0, 0)),
        ],
        out_specs=[
            pl.BlockSpec((_B32, 128), lambda i: (i, 0)),
            pl.BlockSpec((_B32, 128), lambda i: (i, 0)),
            pl.BlockSpec((_B16, 128), lambda i: (i, 0)),
        ],
        out_shape=[
            jax.ShapeDtypeStruct((_P32, 128), jnp.float32),
            jax.ShapeDtypeStruct((_P32, 128), jnp.float32),
            jax.ShapeDtypeStruct((_P16, 128), jnp.float32),
        ],
    )(deg2p, deg2p, xpp, w_bd, me, mo)


def _tmid_same_body(sa_ref, sb_ref, g_ref, dz_ref, b_ref, w_ref, go_ref):
    act = jnp.maximum(
        dz_ref[...] * (sa_ref[0] + sb_ref[0] + g_ref[...]) + b_ref[...], 0.0)
    go_ref[...] = jnp.dot(dz_ref[...] * act, w_ref[...],
                          preferred_element_type=jnp.float32)


def _tmid_same(sp, g, dz, b, w_bd, pin):
    bi = pin // _NBLK
    sp2 = sp.reshape(2, pin, 128)
    return pl.pallas_call(
        _tmid_same_body,
        grid=(_NBLK,),
        in_specs=[
            pl.BlockSpec((1, bi, 128), lambda i: (0, i, 0)),
            pl.BlockSpec((1, bi, 128), lambda i: (1, i, 0)),
            pl.BlockSpec((bi, 128), lambda i: (i, 0)),
            pl.BlockSpec((bi, 128), lambda i: (i, 0)),
            pl.BlockSpec((1, 128), lambda i: (0, 0)),
            pl.BlockSpec((128, 128), lambda i: (0, 0)),
        ],
        out_specs=pl.BlockSpec((bi, 128), lambda i: (i, 0)),
        out_shape=jax.ShapeDtypeStruct((pin, 128), jnp.float32),
    )(sp2, sp2, g, dz, b, w_bd)


def _t3_body(sa_ref, sb_ref, g_ref, dz_ref, b_ref, w_ref, go_ref):
    act = jnp.maximum(
        dz_ref[...] * (sa_ref[0] + sb_ref[0] + g_ref[...]) + b_ref[...], 0.0)
    scaled = (dz_ref[...] * act).reshape(_B16, 2, 128)
    he = jnp.dot(scaled[:, 0, :], w_ref[...],
                 preferred_element_type=jnp.float32)
    ho = jnp.dot(scaled[:, 1, :], w_ref[...],
                 preferred_element_type=jnp.float32)
    go_ref[...] = jnp.concatenate([he, ho], axis=1)


def _t3(sp, g, dz, b, w_bd):
    sp2 = sp.reshape(2, _P32, 128)
    return pl.pallas_call(
        _t3_body,
        grid=(_NBLK,),
        in_specs=[
            pl.BlockSpec((1, _B32, 128), lambda i: (0, i, 0)),
            pl.BlockSpec((1, _B32, 128), lambda i: (1, i, 0)),
            pl.BlockSpec((_B32, 128), lambda i: (i, 0)),
            pl.BlockSpec((_B32, 128), lambda i: (i, 0)),
            pl.BlockSpec((1, 128), lambda i: (0, 0)),
            pl.BlockSpec((128, 64), lambda i: (0, 0)),
        ],
        out_specs=pl.BlockSpec((_B16, 128), lambda i: (i, 0)),
        out_shape=jax.ShapeDtypeStruct((_P16, 128), jnp.float32),
    )(sp2, sp2, g, dz, b, w_bd)


def _tfin_body(sa_ref, sb_ref, g_ref, d16_ref, b_ref, gsum_ref, out_ref):
    z = jnp.maximum(
        d16_ref[...] * (sa_ref[0] + sb_ref[0] + g_ref[...]) + b_ref[...], 0.0)
    col = lax.broadcasted_iota(jnp.int32, z.shape, 1) % 16
    e = jnp.exp(z) * jnp.where(col < 5, 1.0, 0.0)
    denom = jnp.dot(e, gsum_ref[...], preferred_element_type=jnp.float32)
    out_ref[...] = e / denom


def _tfin(sp, g, d16, b, gsum):
    sp2 = sp.reshape(2, _P16, 128)
    return pl.pallas_call(
        _tfin_body,
        grid=(_NBLK,),
        in_specs=[
            pl.BlockSpec((1, _B16, 128), lambda i: (0, i, 0)),
            pl.BlockSpec((1, _B16, 128), lambda i: (1, i, 0)),
            pl.BlockSpec((_B16, 128), lambda i: (i, 0)),
            pl.BlockSpec((_B16, 128), lambda i: (i, 0)),
            pl.BlockSpec((1, 128), lambda i: (0, 0)),
            pl.BlockSpec((128, 128), lambda i: (0, 0)),
        ],
        out_specs=pl.BlockSpec((_B16, 128), lambda i: (i, 0)),
        out_shape=jax.ShapeDtypeStruct((_P16, 128), jnp.float32),
    )(sp2, sp2, g, d16, b, gsum)


def _pad2(a, r, c):
    out = jnp.zeros((r, c), dtype=jnp.float32)
    return out.at[: a.shape[0], : a.shape[1]].set(a)


def kernel(x, edge_index, W1, b1, W3, b3, W4, b4, W5, b5, W2, b2):
    ei = edge_index.astype(jnp.int32)
    pad_ids = _N + (jnp.arange(_EPAD - _E, dtype=jnp.int32) % 128)
    src_p = jnp.concatenate([ei[0], pad_ids])
    dst_p = jnp.concatenate([ei[1], pad_ids])
    src32 = src_p.reshape(-1, _C32)
    dst32 = dst_p.reshape(-1, _C32)
    src16 = src_p.reshape(-1, _C16)
    dst16 = dst_p.reshape(-1, _C16)
    dstdeg = dst_p.reshape(-1, _CDEG)

    xpp = _pad2(x, _NPAD, 32).reshape(_P32, 128)
    w1_bd = jnp.kron(jnp.eye(4, dtype=jnp.float32), _pad2(W1, 32, 32))
    w3_bd = jnp.kron(jnp.eye(4, dtype=jnp.float32), _pad2(W3, 32, 32))
    w4_bd = jnp.kron(jnp.eye(4, dtype=jnp.float32), _pad2(W4, 32, 16))
    w5_bd = jnp.kron(jnp.eye(8, dtype=jnp.float32), _pad2(W5, 16, 16))
    w2_bd = jnp.kron(jnp.eye(8, dtype=jnp.float32), _pad2(W2, 16, 16))
    b1t = jnp.tile(_pad2(b1[None, :], 1, 32), (1, 4))
    b3t = jnp.tile(_pad2(b3[None, :], 1, 32), (1, 4))
    b4t = jnp.tile(_pad2(b4[None, :], 1, 16), (1, 8))
    b5t = jnp.tile(_pad2(b5[None, :], 1, 16), (1, 8))
    b2t = jnp.tile(_pad2(b2[None, :], 1, 16), (1, 8))

    lane = jnp.arange(128)
    m_even = (lane[:, None] == 16 * (lane[None, :] // 32)).astype(jnp.float32)
    m_odd = (lane[:, None] == 64 + 16 * (lane[None, :] // 32)).astype(jnp.float32)
    gsum = (lane[:, None] // 16 == lane[None, :] // 16).astype(jnp.float32)

    zeros32 = jnp.zeros((_NPAD, 32), jnp.float32)
    zeros16 = jnp.zeros((_NPAD, 16), jnp.float32)
    ones16 = jnp.ones((_CDEG, 16), jnp.float32)

    deg2 = _sc_degree(dstdeg, ones16, zeros16)
    g1, d32, d16 = _t1(deg2.reshape(2 * _P16, 128), xpp, w1_bd, m_even, m_odd)
    s1 = _sc_scatter32(g1.reshape(_NPAD, 32), src32, dst32, zeros32)
    g2 = _tmid_same(s1, g1, d32, b1t, w3_bd, _P32)
    s2 = _sc_scatter32(g2.reshape(_NPAD, 32), src32, dst32, zeros32)
    g3 = _t3(s2, g2, d32, b3t, w4_bd)
    s3 = _sc_scatter16(g3.reshape(_NPAD, 16), src16, dst16, zeros16)
    g4 = _tmid_same(s3, g3, d16, b4t, w5_bd, _P16)
    s4 = _sc_scatter16(g4.reshape(_NPAD, 16), src16, dst16, zeros16)
    g5 = _tmid_same(s4, g4, d16, b5t, w2_bd, _P16)
    s5 = _sc_scatter16(g5.reshape(_NPAD, 16), src16, dst16, zeros16)
    p = _tfin(s5, g5, d16, b2t, gsum)
    return p.reshape(_NPAD, 16)[:_N, :5]


# speedup vs baseline: 1.0153x; 1.0050x over previous
import functools

import jax
import jax.numpy as jnp
from jax import lax
from jax.experimental import pallas as pl
from jax.experimental.pallas import tpu as pltpu
from jax.experimental.pallas import tpu_sc as plsc

_N = 50000
_E = 1600000
_NSUB = 16
_NCORE = 2
_NW = _NCORE * _NSUB
_EPW = 51200
_EPAD = _NW * _EPW
_NPAD = 50176
_RPS = _NPAD // _NSUB
_NBLK = 8
_P32 = _NPAD * 32 // 128
_P16 = _NPAD * 16 // 128
_B32 = _P32 // _NBLK
_B16 = _P16 // _NBLK

_C32, _NB32 = 400, 1
_C16, _NB16 = 400, 4
_CDEG, _NBDEG = 800, 4


def _mesh():
    return plsc.VectorSubcoreMesh(core_axis_name="c", subcore_axis_name="s")


_SC_PARAMS = pltpu.CompilerParams(use_tc_tiling_on_sc=False)


@functools.partial(
    pl.kernel,
    out_type=jax.ShapeDtypeStruct((2, _NPAD, 16), jnp.float32),
    mesh=_mesh(),
    scratch_types=[
        pltpu.VMEM((2, _NBDEG, _CDEG), jnp.int32),
        pltpu.VMEM((_CDEG, 16), jnp.float32),
        pltpu.VMEM_SHARED((_NPAD, 16), jnp.float32),
        pltpu.SemaphoreType.DMA,
    ],
    compiler_params=_SC_PARAMS,
)
def _sc_degree(dst_hbm, ones_hbm, zeros_hbm, out_hbm, dst_v, ones_v, acc, ssem):
    cid = lax.axis_index("c")
    sid = lax.axis_index("s")
    wid = cid * _NSUB + sid
    row0 = sid * _RPS
    ch = _EPW // _CDEG
    groups = ch // _NBDEG
    chunk0 = wid * ch
    pltpu.sync_copy(dst_hbm.at[pl.ds(chunk0, _NBDEG)], dst_v.at[0])
    pltpu.sync_copy(ones_hbm, ones_v)
    pltpu.sync_copy(zeros_hbm.at[pl.ds(row0, _RPS)], acc.at[pl.ds(row0, _RPS)])
    plsc.subcore_barrier()

    @pl.loop(0, groups, step=2)
    def _(g0):
        for off in range(2):
            g = g0 + off
            slot, other = off, 1 - off

            @pl.when(g > 0)
            def _():
                for b in range(_NBDEG):
                    pltpu.make_async_copy(
                        ones_v, acc.at[dst_v.at[other, b]], ssem).wait()

            @pl.when(g + 1 < groups)
            def _():
                pltpu.sync_copy(
                    dst_hbm.at[pl.ds(chunk0 + (g + 1) * _NBDEG, _NBDEG)],
                    dst_v.at[other])

            for b in range(_NBDEG):
                pltpu.async_copy(ones_v, acc.at[dst_v.at[slot, b]], ssem, add=True)

    for b in range(_NBDEG):
        pltpu.make_async_copy(ones_v, acc.at[dst_v.at[1, b]], ssem).wait()

    plsc.subcore_barrier()

    @pl.when(cid == 0)
    def _():
        pltpu.sync_copy(acc.at[pl.ds(row0, _RPS)], out_hbm.at[0, pl.ds(row0, _RPS)])

    @pl.when(cid == 1)
    def _():
        pltpu.sync_copy(acc.at[pl.ds(row0, _RPS)], out_hbm.at[1, pl.ds(row0, _RPS)])


def _make_sc_scatter(dp, c, nb):
    ch = _EPW // c
    groups = ch // nb

    @functools.partial(
        pl.kernel,
        out_type=jax.ShapeDtypeStruct((2, _NPAD, dp), jnp.float32),
        mesh=_mesh(),
        scratch_types=[
            pltpu.VMEM((2, nb, c), jnp.int32),
            pltpu.VMEM((2, nb, c), jnp.int32),
            [[pltpu.VMEM((c, dp), jnp.float32) for _ in range(nb)]
             for _ in range(2)],
            pltpu.VMEM_SHARED((_NPAD, dp), jnp.float32),
            pltpu.SemaphoreType.DMA,
            pltpu.SemaphoreType.DMA,
        ],
        compiler_params=_SC_PARAMS,
    )
    def k(g_hbm, src_hbm, dst_hbm, zeros_hbm, out_hbm,
          src_v, dst_v, rows, acc, gsem, ssem):
        cid = lax.axis_index("c")
        sid = lax.axis_index("s")
        wid = cid * _NSUB + sid
        row0 = sid * _RPS
        chunk0 = wid * ch
        pltpu.sync_copy(src_hbm.at[pl.ds(chunk0, nb)], src_v.at[0])
        pltpu.sync_copy(dst_hbm.at[pl.ds(chunk0, nb)], dst_v.at[0])
        for b in range(nb):
            pltpu.async_copy(g_hbm.at[src_v.at[0, b]], rows[0][b], gsem)
        pltpu.sync_copy(zeros_hbm.at[pl.ds(row0, _RPS)], acc.at[pl.ds(row0, _RPS)])
        plsc.subcore_barrier()

        @pl.loop(0, groups, step=2)
        def _(g0):
            for off in range(2):
                g = g0 + off
                slot, other = off, 1 - off

                @pl.when(g > 0)
                def _():
                    for b in range(nb):
                        pltpu.make_async_copy(
                            rows[other][b], acc.at[dst_v.at[other, b]], ssem).wait()

                @pl.when(g + 1 < groups)
                def _():
                    base = chunk0 + (g + 1) * nb
                    pltpu.sync_copy(src_hbm.at[pl.ds(base, nb)], src_v.at[other])
                    pltpu.sync_copy(dst_hbm.at[pl.ds(base, nb)], dst_v.at[other])
                    for b in range(nb):
                        pltpu.async_copy(
                            g_hbm.at[src_v.at[other, b]], rows[other][b], gsem)

                for b in range(nb):
                    pltpu.make_async_copy(
                        g_hbm.at[src_v.at[slot, b]], rows[slot][b], gsem).wait()
                    pltpu.async_copy(
                        rows[slot][b], acc.at[dst_v.at[slot, b]], ssem, add=True)

        for b in range(nb):
            pltpu.make_async_copy(rows[1][b], acc.at[dst_v.at[1, b]], ssem).wait()

        plsc.subcore_barrier()

        @pl.when(cid == 0)
        def _():
            pltpu.sync_copy(acc.at[pl.ds(row0, _RPS)], out_hbm.at[0, pl.ds(row0, _RPS)])

        @pl.when(cid == 1)
        def _():
            pltpu.sync_copy(acc.at[pl.ds(row0, _RPS)], out_hbm.at[1, pl.ds(row0, _RPS)])

    return k


_sc_scatter32 = _make_sc_scatter(32, _C32, _NB32)
_sc_scatter16 = _make_sc_scatter(16, _C16, _NB16)


def _t1_body(da_ref, db_ref, xp_ref, w_ref, me_ref, mo_ref,
             g_ref, d32_ref, d16_ref):
    dinv16 = lax.rsqrt(da_ref[...] + db_ref[...] + 1.0)
    d16_ref[...] = dinv16
    de = jnp.dot(dinv16, me_ref[...], preferred_element_type=jnp.float32)
    do = jnp.dot(dinv16, mo_ref[...], preferred_element_type=jnp.float32)
    dinv32 = jnp.stack([de, do], axis=1).reshape(_B32, 128)
    d32_ref[...] = dinv32
    h = jnp.dot(xp_ref[...], w_ref[...], preferred_element_type=jnp.float32)
    g_ref[...] = dinv32 * h


def _t1(deg2p, xpp, w_bd, me, mo):
    return pl.pallas_call(
        _t1_body,
        grid=(_NBLK,),
        in_specs=[
            pl.BlockSpec((_B16, 128), lambda i: (i, 0)),
            pl.BlockSpec((_B16, 128), lambda i: (i + _NBLK, 0)),
            pl.BlockSpec((_B32, 128), lambda i: (i, 0)),
            pl.BlockSpec((128, 128), lambda i: (0, 0)),
            pl.BlockSpec((128, 128), lambda i: (0, 0)),
            pl.BlockSpec((128, 128), lambda i: (0, 0)),
        ],
        out_specs=[
            pl.BlockSpec((_B32, 128), lambda i: (i, 0)),
            pl.BlockSpec((_B32, 128), lambda i: (i, 0)),
            pl.BlockSpec((_B16, 128), lambda i: (i, 0)),
        ],
        out_shape=[
            jax.ShapeDtypeStruct((_P32, 128), jnp.float32),
            jax.ShapeDtypeStruct((_P32, 128), jnp.float32),
            jax.ShapeDtypeStruct((_P16, 128), jnp.float32),
        ],
    )(deg2p, deg2p, xpp, w_bd, me, mo)


def _tmid_same_body(sa_ref, sb_ref, g_ref, dz_ref, b_ref, w_ref, go_ref):
    act = jnp.maximum(
        dz_ref[...] * (sa_ref[0] + sb_ref[0] + g_ref[...]) + b_ref[...], 0.0)
    go_ref[...] = jnp.dot(dz_ref[...] * act, w_ref[...],
                          preferred_element_type=jnp.float32)


def _tmid_same(sp, g, dz, b, w_bd, pin):
    bi = pin // _NBLK
    sp2 = sp.reshape(2, pin, 128)
    return pl.pallas_call(
        _tmid_same_body,
        grid=(_NBLK,),
        in_specs=[
            pl.BlockSpec((1, bi, 128), lambda i: (0, i, 0)),
            pl.BlockSpec((1, bi, 128), lambda i: (1, i, 0)),
            pl.BlockSpec((bi, 128), lambda i: (i, 0)),
            pl.BlockSpec((bi, 128), lambda i: (i, 0)),
            pl.BlockSpec((1, 128), lambda i: (0, 0)),
            pl.BlockSpec((128, 128), lambda i: (0, 0)),
        ],
        out_specs=pl.BlockSpec((bi, 128), lambda i: (i, 0)),
        out_shape=jax.ShapeDtypeStruct((pin, 128), jnp.float32),
    )(sp2, sp2, g, dz, b, w_bd)


def _t3_body(sa_ref, sb_ref, g_ref, dz_ref, b_ref, w_ref, go_ref):
    act = jnp.maximum(
        dz_ref[...] * (sa_ref[0] + sb_ref[0] + g_ref[...]) + b_ref[...], 0.0)
    scaled = (dz_ref[...] * act).reshape(_B16, 2, 128)
    he = jnp.dot(scaled[:, 0, :], w_ref[...],
                 preferred_element_type=jnp.float32)
    ho = jnp.dot(scaled[:, 1, :], w_ref[...],
                 preferred_element_type=jnp.float32)
    go_ref[...] = jnp.concatenate([he, ho], axis=1)


def _t3(sp, g, dz, b, w_bd):
    sp2 = sp.reshape(2, _P32, 128)
    return pl.pallas_call(
        _t3_body,
        grid=(_NBLK,),
        in_specs=[
            pl.BlockSpec((1, _B32, 128), lambda i: (0, i, 0)),
            pl.BlockSpec((1, _B32, 128), lambda i: (1, i, 0)),
            pl.BlockSpec((_B32, 128), lambda i: (i, 0)),
            pl.BlockSpec((_B32, 128), lambda i: (i, 0)),
            pl.BlockSpec((1, 128), lambda i: (0, 0)),
            pl.BlockSpec((128, 64), lambda i: (0, 0)),
        ],
        out_specs=pl.BlockSpec((_B16, 128), lambda i: (i, 0)),
        out_shape=jax.ShapeDtypeStruct((_P16, 128), jnp.float32),
    )(sp2, sp2, g, dz, b, w_bd)


def _tfin_body(sa_ref, sb_ref, g_ref, d16_ref, b_ref, gsum_ref, out_ref):
    z = jnp.maximum(
        d16_ref[...] * (sa_ref[0] + sb_ref[0] + g_ref[...]) + b_ref[...], 0.0)
    col = lax.broadcasted_iota(jnp.int32, z.shape, 1) % 16
    e = jnp.exp(z) * jnp.where(col < 5, 1.0, 0.0)
    denom = jnp.dot(e, gsum_ref[...], preferred_element_type=jnp.float32)
    out_ref[...] = e / denom


def _tfin(sp, g, d16, b, gsum):
    sp2 = sp.reshape(2, _P16, 128)
    return pl.pallas_call(
        _tfin_body,
        grid=(_NBLK,),
        in_specs=[
            pl.BlockSpec((1, _B16, 128), lambda i: (0, i, 0)),
            pl.BlockSpec((1, _B16, 128), lambda i: (1, i, 0)),
            pl.BlockSpec((_B16, 128), lambda i: (i, 0)),
            pl.BlockSpec((_B16, 128), lambda i: (i, 0)),
            pl.BlockSpec((1, 128), lambda i: (0, 0)),
            pl.BlockSpec((128, 128), lambda i: (0, 0)),
        ],
        out_specs=pl.BlockSpec((_B16, 128), lambda i: (i, 0)),
        out_shape=jax.ShapeDtypeStruct((_P16, 128), jnp.float32),
    )(sp2, sp2, g, d16, b, gsum)


def _pad2(a, r, c):
    out = jnp.zeros((r, c), dtype=jnp.float32)
    return out.at[: a.shape[0], : a.shape[1]].set(a)


def kernel(x, edge_index, W1, b1, W3, b3, W4, b4, W5, b5, W2, b2):
    ei = edge_index.astype(jnp.int32)
    pad_ids = _N + (jnp.arange(_EPAD - _E, dtype=jnp.int32) % 128)
    src_p = jnp.concatenate([ei[0], pad_ids])
    dst_p = jnp.concatenate([ei[1], pad_ids])
    src32 = src_p.reshape(-1, _C32)
    dst32 = dst_p.reshape(-1, _C32)
    src16 = src_p.reshape(-1, _C16)
    dst16 = dst_p.reshape(-1, _C16)
    dstdeg = dst_p.reshape(-1, _CDEG)

    xpp = _pad2(x, _NPAD, 32).reshape(_P32, 128)
    w1_bd = jnp.kron(jnp.eye(4, dtype=jnp.float32), _pad2(W1, 32, 32))
    w3_bd = jnp.kron(jnp.eye(4, dtype=jnp.float32), _pad2(W3, 32, 32))
    w4_bd = jnp.kron(jnp.eye(4, dtype=jnp.float32), _pad2(W4, 32, 16))
    w5_bd = jnp.kron(jnp.eye(8, dtype=jnp.float32), _pad2(W5, 16, 16))
    w2_bd = jnp.kron(jnp.eye(8, dtype=jnp.float32), _pad2(W2, 16, 16))
    b1t = jnp.tile(_pad2(b1[None, :], 1, 32), (1, 4))
    b3t = jnp.tile(_pad2(b3[None, :], 1, 32), (1, 4))
    b4t = jnp.tile(_pad2(b4[None, :], 1, 16), (1, 8))
    b5t = jnp.tile(_pad2(b5[None, :], 1, 16), (1, 8))
    b2t = jnp.tile(_pad2(b2[None, :], 1, 16), (1, 8))

    lane = jnp.arange(128)
    m_even = (lane[:, None] == 16 * (lane[None, :] // 32)).astype(jnp.float32)
    m_odd = (lane[:, None] == 64 + 16 * (lane[None, :] // 32)).astype(jnp.float32)
    gsum = (lane[:, None] // 16 == lane[None, :] // 16).astype(jnp.float32)

    zeros32 = jnp.zeros((_NPAD, 32), jnp.float32)
    zeros16 = jnp.zeros((_NPAD, 16), jnp.float32)
    ones16 = jnp.ones((_CDEG, 16), jnp.float32)

    deg2 = _sc_degree(dstdeg, ones16, zeros16)
    g1, d32, d16 = _t1(deg2.reshape(2 * _P16, 128), xpp, w1_bd, m_even, m_odd)
    s1 = _sc_scatter32(g1.reshape(_NPAD, 32), src32, dst32, zeros32)
    g2 = _tmid_same(s1, g1, d32, b1t, w3_bd, _P32)
    s2 = _sc_scatter32(g2.reshape(_NPAD, 32), src32, dst32, zeros32)
    g3 = _t3(s2, g2, d32, b3t, w4_bd)
    s3 = _sc_scatter16(g3.reshape(_NPAD, 16), src16, dst16, zeros16)
    g4 = _tmid_same(s3, g3, d16, b4t, w5_bd, _P16)
    s4 = _sc_scatter16(g4.reshape(_NPAD, 16), src16, dst16, zeros16)
    g5 = _tmid_same(s4, g4, d16, b5t, w2_bd, _P16)
    s5 = _sc_scatter16(g5.reshape(_NPAD, 16), src16, dst16, zeros16)
    p = _tfin(s5, g5, d16, b2t, gsum)
    return p.reshape(_NPAD, 16)[:_N, :5]
